# baseline (speedup 1.0000x reference)
import functools

import jax
import jax.numpy as jnp
from jax import lax
from jax.experimental import pallas as pl
from jax.experimental.pallas import tpu as pltpu
from jax.experimental.pallas import tpu_sc as plsc

NC = 2
NS = 16
L = 16

M, D, B = 100000, 128, 16384
NBLK = 20
R = M // NBLK
BPC = NBLK // NC
RPT = 312
RLAST = R - (NS - 1) * RPT
BPT = B // NS
NV = BPT // L
TRASH = R
G = 128


def _sc_body(in_hbm, idx_hbm, val_hbm, out_hbm,
             my_idx, lb0, pb0, lb1, pb1, loc128, vbuf,
             blk0, blk1, gg, gs, si0, si1, so0, so1):
    c = lax.axis_index("c")
    s = lax.axis_index("s")

    blks = [blk0, blk1]
    sins = [si0, si1]
    souts = [so0, so1]
    lbs = [lb0, lb1]
    pbs = [pb0, pb1]

    def blk_base(i):
        return (NC * i + c) * R

    def copy_in(i, sem, start):
        base = blk_base(i)
        buf = blks[i % 2]

        @pl.when(s < NS - 1)
        def _():
            cp = pltpu.make_async_copy(
                in_hbm.at[pl.ds(base + s * RPT, RPT)],
                buf.at[pl.ds(s * RPT, RPT)], sem)
            cp.start() if start else cp.wait()

        @pl.when(s == NS - 1)
        def _():
            cp = pltpu.make_async_copy(
                in_hbm.at[pl.ds(base + (NS - 1) * RPT, RLAST)],
                buf.at[pl.ds((NS - 1) * RPT, RLAST)], sem)
            cp.start() if start else cp.wait()

    def copy_out(i, sem, start):
        base = blk_base(i)
        buf = blks[i % 2]

        @pl.when(s < NS - 1)
        def _():
            cp = pltpu.make_async_copy(
                buf.at[pl.ds(s * RPT, RPT)],
                out_hbm.at[pl.ds(base + s * RPT, RPT)], sem)
            cp.start() if start else cp.wait()

        @pl.when(s == NS - 1)
        def _():
            cp = pltpu.make_async_copy(
                buf.at[pl.ds((NS - 1) * RPT, RLAST)],
                out_hbm.at[pl.ds(base + (NS - 1) * RPT, RLAST)], sem)
            cp.start() if start else cp.wait()

    def compact(i, lb, pb):
        base = blk_base(i)

        def cbody(j, cnt):
            iv = my_idx[pl.ds(j * L, L)]
            basev = jnp.full((L,), base, jnp.int32)
            limv = jnp.full((L,), base + R, jnp.int32)
            m = (iv >= basev) & (iv < limv)
            loc = iv - basev
            pos = jnp.full((L,), s * BPT + j * L, jnp.int32) + lax.iota(jnp.int32, L)
            mi = jnp.where(m, 1, 0).astype(jnp.int32)
            pc = plsc.cumsum(mi)
            dest = jnp.full((L,), cnt, jnp.int32) + pc - 1
            plsc.store_scatter(lb, [dest], loc, mask=m)
            plsc.store_scatter(pb, [dest], pos, mask=m)
            return cnt + jnp.sum(mi)

        cnt = lax.fori_loop(0, NV, cbody, jnp.int32(0))

        vstart = (cnt // L) * L
        lane = jnp.full((L,), vstart, jnp.int32) + lax.iota(jnp.int32, L)
        valid = lane < jnp.full((L,), cnt, jnp.int32)
        lv = lb[pl.ds(vstart, L)]
        pv = pb[pl.ds(vstart, L)]
        lb[pl.ds(vstart, L)] = jnp.where(valid, lv, TRASH)
        pb[pl.ds(vstart, L)] = jnp.where(valid, pv, 0)

        def padbody(j, off):
            lb[pl.ds(off, L)] = jnp.full((L,), TRASH, jnp.int32)
            pb[pl.ds(off, L)] = jnp.zeros((L,), jnp.int32)
            return off + L

        npad = (G // L) - 1 - lax.rem(cnt // L, G // L)
        lax.fori_loop(0, npad, padbody, vstart + L)
        return cnt

    def fire_g(pb, lb, off):
        pltpu.async_copy(val_hbm.at[pb.at[pl.ds(off, G)]], vbuf, gg)
        for t in range(G // L):
            loc128[pl.ds(t * L, L)] = lb[pl.ds(off + t * L, L)]

    def drain_g(pb, off):
        pltpu.make_async_copy(val_hbm.at[pb.at[pl.ds(off, G)]], vbuf, gg).wait()

    def fire_s(blk):
        pltpu.async_copy(vbuf, blk.at[loc128], gs, add=True)

    def drain_s(blk):
        pltpu.make_async_copy(vbuf, blk.at[loc128], gs).wait()

    pltpu.sync_copy(idx_hbm.at[pl.ds(s * BPT, BPT)], my_idx)
    copy_in(0, sins[0], True)
    cnt = compact(0, lbs[0], pbs[0])
    copy_in(0, sins[0], False)
    plsc.subcore_barrier()

    for i in range(BPC):
        q = i % 2
        blk = blks[q]
        npass = (cnt + G - 1) // G
        live = cnt > 0

        with jax.named_scope("dg"):
            @pl.when(live)
            def _():
                fire_g(pbs[q], lbs[q], 0)

            if i + 1 < BPC:
                if i >= 1:
                    copy_out(i - 1, souts[1 - q], False)
                copy_in(i + 1, sins[1 - q], True)
                cnt1 = compact(i + 1, lbs[1 - q], pbs[1 - q])

            @pl.when(live)
            def _():
                drain_g(pbs[q], 0)
                fire_s(blk)

        if i + 1 < BPC:
            with jax.named_scope("cinw"):
                copy_in(i + 1, sins[1 - q], False)

        with jax.named_scope("ds"):
            @pl.when(live)
            def _():
                drain_s(blk)

        def pass_body(ps, carry):
            fire_g(pbs[q], lbs[q], ps * G)
            drain_g(pbs[q], ps * G)
            fire_s(blk)
            drain_s(blk)
            return carry

        lax.fori_loop(1, npass, pass_body, jnp.int32(0))

        if i + 1 < BPC:
            cnt = cnt1

        with jax.named_scope("bar"):
            plsc.subcore_barrier()
        copy_out(i, souts[q], True)

    copy_out(BPC - 2, souts[(BPC - 2) % 2], False)
    copy_out(BPC - 1, souts[(BPC - 1) % 2], False)


@jax.jit
def _scatter_add(input, idx32, value):
    kfn = functools.partial(
        pl.kernel,
        mesh=plsc.VectorSubcoreMesh(core_axis_name="c", subcore_axis_name="s"),
        out_type=jax.ShapeDtypeStruct((M, D), jnp.float32),
        scratch_types=[
            pltpu.VMEM((BPT,), jnp.int32),
            pltpu.VMEM((BPT + 2 * G,), jnp.int32),
            pltpu.VMEM((BPT + 2 * G,), jnp.int32),
            pltpu.VMEM((BPT + 2 * G,), jnp.int32),
            pltpu.VMEM((BPT + 2 * G,), jnp.int32),
            pltpu.VMEM((G,), jnp.int32),
            pltpu.VMEM((G, D), jnp.float32),
            pltpu.VMEM_SHARED((R + L, D), jnp.float32),
            pltpu.VMEM_SHARED((R + L, D), jnp.float32),
            pltpu.SemaphoreType.DMA,
            pltpu.SemaphoreType.DMA,
            pltpu.SemaphoreType.DMA,
            pltpu.SemaphoreType.DMA,
            pltpu.SemaphoreType.DMA,
            pltpu.SemaphoreType.DMA,
        ],
        compiler_params=pltpu.CompilerParams(needs_layout_passes=False),
    )(_sc_body)
    return kfn(input, idx32, value)


def kernel(input, index, value):
    assert input.shape == (M, D) and value.shape == (B, D)
    return _scatter_add(input, index.astype(jnp.int32), value)


# speedup vs baseline: 11.8525x; 11.8525x over previous
import functools

import jax
import jax.numpy as jnp
from jax import lax
from jax.experimental import pallas as pl
from jax.experimental.pallas import tpu as pltpu
from jax.experimental.pallas import tpu_sc as plsc

NC = 2
NS = 16
L = 16

M, D, B = 100000, 128, 16384
NBLK = 20
R = M // NBLK
BPC = NBLK // NC
RPT = 312
RLAST = R - (NS - 1) * RPT
BPT = B // NS
NV = BPT // L
TRASH = R
G = 128


def _sc_body(in_hbm, idx_hbm, val_hbm, out_hbm,
             my_idx, lb0, pb0, lb1, pb1, locs2d, vbuf,
             blk0, blk1, gg, gs, si0, si1, so0, so1):
    c = lax.axis_index("c")
    s = lax.axis_index("s")

    blks = [blk0, blk1]
    sins = [si0, si1]
    souts = [so0, so1]
    lbs = [lb0, lb1]
    pbs = [pb0, pb1]

    def blk_base(i):
        return (NC * i + c) * R

    def copy_in(i, sem, start):
        base = blk_base(i)
        buf = blks[i % 2]

        @pl.when(s < NS - 1)
        def _():
            cp = pltpu.make_async_copy(
                in_hbm.at[pl.ds(base + s * RPT, RPT)],
                buf.at[pl.ds(s * RPT, RPT)], sem)
            cp.start() if start else cp.wait()

        @pl.when(s == NS - 1)
        def _():
            cp = pltpu.make_async_copy(
                in_hbm.at[pl.ds(base + (NS - 1) * RPT, RLAST)],
                buf.at[pl.ds((NS - 1) * RPT, RLAST)], sem)
            cp.start() if start else cp.wait()

    def copy_out(i, sem, start):
        base = blk_base(i)
        buf = blks[i % 2]

        @pl.when(s < NS - 1)
        def _():
            cp = pltpu.make_async_copy(
                buf.at[pl.ds(s * RPT, RPT)],
                out_hbm.at[pl.ds(base + s * RPT, RPT)], sem)
            cp.start() if start else cp.wait()

        @pl.when(s == NS - 1)
        def _():
            cp = pltpu.make_async_copy(
                buf.at[pl.ds((NS - 1) * RPT, RLAST)],
                out_hbm.at[pl.ds(base + (NS - 1) * RPT, RLAST)], sem)
            cp.start() if start else cp.wait()

    def compact(i, lb, pb):
        base = blk_base(i)

        def cbody(j, cnt):
            iv = my_idx[pl.ds(j * L, L)]
            basev = jnp.full((L,), base, jnp.int32)
            limv = jnp.full((L,), base + R, jnp.int32)
            m = (iv >= basev) & (iv < limv)
            loc = iv - basev
            pos = jnp.full((L,), s * BPT + j * L, jnp.int32) + lax.iota(jnp.int32, L)
            mi = jnp.where(m, 1, 0).astype(jnp.int32)
            pc = plsc.cumsum(mi)
            dest = jnp.full((L,), cnt, jnp.int32) + pc - 1
            plsc.store_scatter(lb, [dest], loc, mask=m)
            plsc.store_scatter(pb, [dest], pos, mask=m)
            return cnt + jnp.sum(mi)

        cnt = lax.fori_loop(0, NV, cbody, jnp.int32(0))

        vstart = (cnt // L) * L
        iota = lax.iota(jnp.int32, L)
        lane = jnp.full((L,), vstart, jnp.int32) + iota
        valid = lane < jnp.full((L,), cnt, jnp.int32)
        lv = lb[pl.ds(vstart, L)]
        pv = pb[pl.ds(vstart, L)]
        lb[pl.ds(vstart, L)] = jnp.where(valid, lv, jnp.full((L,), TRASH, jnp.int32) + iota)
        pb[pl.ds(vstart, L)] = jnp.where(valid, pv, jnp.full((L,), s * BPT, jnp.int32) + iota)
        return cnt

    GQ = G // L

    def fire_g(pb, lb, off, u):
        for j in range(GQ):
            @pl.when(j < u)
            def _(j=j):
                pltpu.async_copy(
                    val_hbm.at[pb.at[pl.ds(off + j * L, L)]],
                    vbuf.at[pl.ds(j * L, L)], gg)
                locs2d[j, :] = lb[pl.ds(off + j * L, L)]

    def drain_g(pb, off, u):
        for j in range(GQ):
            @pl.when(j < u)
            def _(j=j):
                pltpu.make_async_copy(
                    val_hbm.at[pb.at[pl.ds(off + j * L, L)]],
                    vbuf.at[pl.ds(j * L, L)], gg).wait()

    def fire_s(blk, u):
        for j in range(GQ):
            @pl.when(j < u)
            def _(j=j):
                pltpu.async_copy(
                    vbuf.at[pl.ds(j * L, L)],
                    blk.at[locs2d.at[j]], gs, add=True)

    def drain_s(blk, u):
        for j in range(GQ):
            @pl.when(j < u)
            def _(j=j):
                pltpu.make_async_copy(
                    vbuf.at[pl.ds(j * L, L)],
                    blk.at[locs2d.at[j]], gs).wait()

    pltpu.sync_copy(idx_hbm.at[pl.ds(s * BPT, BPT)], my_idx)
    copy_in(0, sins[0], True)
    cnt = compact(0, lbs[0], pbs[0])
    copy_in(0, sins[0], False)
    plsc.subcore_barrier()

    for i in range(BPC):
        q = i % 2
        blk = blks[q]
        units = (cnt + L - 1) // L
        npass = (units + GQ - 1) // GQ
        u0 = jnp.minimum(units, GQ)

        with jax.named_scope("dg"):
            fire_g(pbs[q], lbs[q], 0, u0)

            if i + 1 < BPC:
                if i >= 1:
                    copy_out(i - 1, souts[1 - q], False)
                copy_in(i + 1, sins[1 - q], True)
                cnt1 = compact(i + 1, lbs[1 - q], pbs[1 - q])

            drain_g(pbs[q], 0, u0)
            fire_s(blk, u0)

        if i + 1 < BPC:
            with jax.named_scope("cinw"):
                copy_in(i + 1, sins[1 - q], False)

        with jax.named_scope("ds"):
            drain_s(blk, u0)

        def pass_body(ps, carry):
            ux = jnp.minimum(units - ps * GQ, GQ)
            fire_g(pbs[q], lbs[q], ps * G, ux)
            drain_g(pbs[q], ps * G, ux)
            fire_s(blk, ux)
            drain_s(blk, ux)
            return carry

        lax.fori_loop(1, npass, pass_body, jnp.int32(0))

        if i + 1 < BPC:
            cnt = cnt1

        with jax.named_scope("bar"):
            plsc.subcore_barrier()
        copy_out(i, souts[q], True)

    copy_out(BPC - 2, souts[(BPC - 2) % 2], False)
    copy_out(BPC - 1, souts[(BPC - 1) % 2], False)


@jax.jit
def _scatter_add(input, idx32, value):
    kfn = functools.partial(
        pl.kernel,
        mesh=plsc.VectorSubcoreMesh(core_axis_name="c", subcore_axis_name="s"),
        out_type=jax.ShapeDtypeStruct((M, D), jnp.float32),
        scratch_types=[
            pltpu.VMEM((BPT,), jnp.int32),
            pltpu.VMEM((BPT + 2 * G,), jnp.int32),
            pltpu.VMEM((BPT + 2 * G,), jnp.int32),
            pltpu.VMEM((BPT + 2 * G,), jnp.int32),
            pltpu.VMEM((BPT + 2 * G,), jnp.int32),
            pltpu.VMEM((G // L, L), jnp.int32),
            pltpu.VMEM((G, D), jnp.float32),
            pltpu.VMEM_SHARED((R + L, D), jnp.float32),
            pltpu.VMEM_SHARED((R + L, D), jnp.float32),
            pltpu.SemaphoreType.DMA,
            pltpu.SemaphoreType.DMA,
            pltpu.SemaphoreType.DMA,
            pltpu.SemaphoreType.DMA,
            pltpu.SemaphoreType.DMA,
            pltpu.SemaphoreType.DMA,
        ],
        compiler_params=pltpu.CompilerParams(needs_layout_passes=False),
    )(_sc_body)
    return kfn(input, idx32, value)


def kernel(input, index, value):
    assert input.shape == (M, D) and value.shape == (B, D)
    return _scatter_add(input, index.astype(jnp.int32), value)


# speedup vs baseline: 11.8741x; 1.0018x over previous
import functools

import jax
import jax.numpy as jnp
from jax import lax
from jax.experimental import pallas as pl
from jax.experimental.pallas import tpu as pltpu
from jax.experimental.pallas import tpu_sc as plsc

NC = 2
NS = 16
L = 16

M, D, B = 100000, 128, 16384
NBLK = 20
R = M // NBLK
BPC = NBLK // NC
RPT = 312
RLAST = R - (NS - 1) * RPT
BPT = B // NS
NV = BPT // L
TRASH = R
G = 128


def _sc_body(in_hbm, idx_hbm, val_hbm, out_hbm,
             my_idx, lb0, pb0, lb1, pb1, locs2d, vbuf,
             blk0, blk1, gg, gs, si0, si1, so0, so1):
    c = lax.axis_index("c")
    s = lax.axis_index("s")

    blks = [blk0, blk1]
    sins = [si0, si1]
    souts = [so0, so1]
    lbs = [lb0, lb1]
    pbs = [pb0, pb1]

    def blk_base(i):
        return (NC * i + c) * R

    def copy_in(i, sem, start):
        base = blk_base(i)
        buf = blks[i % 2]

        @pl.when(s < NS - 1)
        def _():
            cp = pltpu.make_async_copy(
                in_hbm.at[pl.ds(base + s * RPT, RPT)],
                buf.at[pl.ds(s * RPT, RPT)], sem)
            cp.start() if start else cp.wait()

        @pl.when(s == NS - 1)
        def _():
            cp = pltpu.make_async_copy(
                in_hbm.at[pl.ds(base + (NS - 1) * RPT, RLAST)],
                buf.at[pl.ds((NS - 1) * RPT, RLAST)], sem)
            cp.start() if start else cp.wait()

    def copy_out(i, sem, start):
        base = blk_base(i)
        buf = blks[i % 2]

        @pl.when(s < NS - 1)
        def _():
            cp = pltpu.make_async_copy(
                buf.at[pl.ds(s * RPT, RPT)],
                out_hbm.at[pl.ds(base + s * RPT, RPT)], sem)
            cp.start() if start else cp.wait()

        @pl.when(s == NS - 1)
        def _():
            cp = pltpu.make_async_copy(
                buf.at[pl.ds((NS - 1) * RPT, RLAST)],
                out_hbm.at[pl.ds(base + (NS - 1) * RPT, RLAST)], sem)
            cp.start() if start else cp.wait()

    def compact(i, lb, pb):
        base = blk_base(i)

        def cbody(j, cnt):
            iv = my_idx[pl.ds(j * L, L)]
            basev = jnp.full((L,), base, jnp.int32)
            limv = jnp.full((L,), base + R, jnp.int32)
            m = (iv >= basev) & (iv < limv)
            loc = iv - basev
            pos = jnp.full((L,), s * BPT + j * L, jnp.int32) + lax.iota(jnp.int32, L)
            mi = jnp.where(m, 1, 0).astype(jnp.int32)
            pc = plsc.cumsum(mi)
            dest = jnp.full((L,), cnt, jnp.int32) + pc - 1
            plsc.store_scatter(lb, [dest], loc, mask=m)
            plsc.store_scatter(pb, [dest], pos, mask=m)
            return cnt + jnp.sum(mi)

        cnt = lax.fori_loop(0, NV, cbody, jnp.int32(0))

        vstart = (cnt // L) * L
        iota = lax.iota(jnp.int32, L)
        lane = jnp.full((L,), vstart, jnp.int32) + iota
        valid = lane < jnp.full((L,), cnt, jnp.int32)
        lv = lb[pl.ds(vstart, L)]
        pv = pb[pl.ds(vstart, L)]
        lb[pl.ds(vstart, L)] = jnp.where(valid, lv, jnp.full((L,), TRASH, jnp.int32) + iota)
        pb[pl.ds(vstart, L)] = jnp.where(valid, pv, jnp.full((L,), s * BPT, jnp.int32) + iota)
        return cnt

    GQ = G // L

    def fire_g(pb, lb, off, u):
        for j in range(GQ):
            @pl.when(j < u)
            def _(j=j):
                pltpu.async_copy(
                    val_hbm.at[pb.at[pl.ds(off + j * L, L)]],
                    vbuf.at[pl.ds(j * L, L)], gg)
                locs2d[j, :] = lb[pl.ds(off + j * L, L)]

    def drain_g(pb, off, u):
        for j in range(GQ):
            @pl.when(j < u)
            def _(j=j):
                pltpu.make_async_copy(
                    val_hbm.at[pb.at[pl.ds(off + j * L, L)]],
                    vbuf.at[pl.ds(j * L, L)], gg).wait()

    def fire_s(blk, u):
        for j in range(GQ):
            @pl.when(j < u)
            def _(j=j):
                pltpu.async_copy(
                    vbuf.at[pl.ds(j * L, L)],
                    blk.at[locs2d.at[j]], gs, add=True)

    def drain_s(blk, u):
        for j in range(GQ):
            @pl.when(j < u)
            def _(j=j):
                pltpu.make_async_copy(
                    vbuf.at[pl.ds(j * L, L)],
                    blk.at[locs2d.at[j]], gs).wait()

    pltpu.sync_copy(idx_hbm.at[pl.ds(s * BPT, BPT)], my_idx)
    copy_in(0, sins[0], True)
    cnt = compact(0, lbs[0], pbs[0])
    copy_in(0, sins[0], False)
    plsc.subcore_barrier()

    for i in range(BPC):
        q = i % 2
        blk = blks[q]
        units = (cnt + L - 1) // L
        npass = (units + GQ - 1) // GQ
        u0 = jnp.minimum(units, GQ)

        fire_g(pbs[q], lbs[q], 0, u0)

        if i + 1 < BPC:
            if i >= 1:
                copy_out(i - 1, souts[1 - q], False)
            copy_in(i + 1, sins[1 - q], True)
            cnt1 = compact(i + 1, lbs[1 - q], pbs[1 - q])

        drain_g(pbs[q], 0, u0)
        fire_s(blk, u0)

        if i + 1 < BPC:
            copy_in(i + 1, sins[1 - q], False)

        drain_s(blk, u0)

        def pass_body(ps, carry):
            ux = jnp.minimum(units - ps * GQ, GQ)
            fire_g(pbs[q], lbs[q], ps * G, ux)
            drain_g(pbs[q], ps * G, ux)
            fire_s(blk, ux)
            drain_s(blk, ux)
            return carry

        lax.fori_loop(1, npass, pass_body, jnp.int32(0))

        if i + 1 < BPC:
            cnt = cnt1

        plsc.subcore_barrier()
        copy_out(i, souts[q], True)

    copy_out(BPC - 2, souts[(BPC - 2) % 2], False)
    copy_out(BPC - 1, souts[(BPC - 1) % 2], False)


@jax.jit
def _scatter_add(input, idx32, value):
    kfn = functools.partial(
        pl.kernel,
        mesh=plsc.VectorSubcoreMesh(core_axis_name="c", subcore_axis_name="s"),
        out_type=jax.ShapeDtypeStruct((M, D), jnp.float32),
        scratch_types=[
            pltpu.VMEM((BPT,), jnp.int32),
            pltpu.VMEM((BPT + 2 * G,), jnp.int32),
            pltpu.VMEM((BPT + 2 * G,), jnp.int32),
            pltpu.VMEM((BPT + 2 * G,), jnp.int32),
            pltpu.VMEM((BPT + 2 * G,), jnp.int32),
            pltpu.VMEM((G // L, L), jnp.int32),
            pltpu.VMEM((G, D), jnp.float32),
            pltpu.VMEM_SHARED((R + L, D), jnp.float32),
            pltpu.VMEM_SHARED((R + L, D), jnp.float32),
            pltpu.SemaphoreType.DMA,
            pltpu.SemaphoreType.DMA,
            pltpu.SemaphoreType.DMA,
            pltpu.SemaphoreType.DMA,
            pltpu.SemaphoreType.DMA,
            pltpu.SemaphoreType.DMA,
        ],
        compiler_params=pltpu.CompilerParams(needs_layout_passes=False),
    )(_sc_body)
    return kfn(input, idx32, value)


def kernel(input, index, value):
    assert input.shape == (M, D) and value.shape == (B, D)
    return _scatter_add(input, index.astype(jnp.int32), value)
